# baseline (speedup 1.0000x reference)
import functools

import jax
import jax.numpy as jnp
from jax import lax
from jax.experimental import pallas as pl
from jax.experimental.pallas import tpu as pltpu
from jax.experimental.pallas import tpu_sc as plsc

NC = 2
NS = 16
NW = NC * NS
CHUNK = 128

NODE_BLK = 2000
EDGE_BLK = 1280


def _a_body(x_ref, w_ref, o_ref):
    o_ref[...] = jnp.dot(x_ref[...], w_ref[0], preferred_element_type=jnp.float32)


def _stage_a(x, w_rs):
    n, h = x.shape
    nb = n // NODE_BLK
    return pl.pallas_call(
        _a_body,
        grid=(2, nb),
        in_specs=[
            pl.BlockSpec((NODE_BLK, h), lambda j, i: (i, 0)),
            pl.BlockSpec((1, h, h), lambda j, i: (j, 0, 0)),
        ],
        out_specs=pl.BlockSpec((NODE_BLK, h), lambda j, i: (j * nb + i, 0)),
        out_shape=jax.ShapeDtypeStruct((2 * n, h), jnp.float32),
    )(x, w_rs)


def _pad_blocks(nblk):
    per_tile = -(-nblk // (NW * 8)) * 8
    return per_tile


def _gather_body(cnt, tab_ref, ridx_ref, sidx_ref, out_ref,
                 idxr_v, idxs_v, rows_r, rows_s, sem_a, sem_b):
    h = tab_ref.shape[1]
    wid = lax.axis_index("s") * NC + lax.axis_index("c")
    start = pl.multiple_of(wid * cnt, 8)

    pltpu.sync_copy(ridx_ref.at[pl.ds(start, cnt)], idxr_v)
    pltpu.sync_copy(sidx_ref.at[pl.ds(start, cnt)], idxs_v)

    pltpu.async_copy(tab_ref.at[idxr_v.at[0]], rows_r.at[0], sem_a)
    pltpu.async_copy(tab_ref.at[idxs_v.at[0]], rows_s.at[0], sem_a)

    def step(c, bcur, sem_cur, sem_nxt, issue_next):
        bnxt = 1 - bcur

        pltpu.make_async_copy(tab_ref.at[idxr_v.at[0]], rows_r.at[bcur], sem_cur).wait()
        pltpu.make_async_copy(tab_ref.at[idxs_v.at[0]], rows_s.at[bcur], sem_cur).wait()

        if issue_next:
            pltpu.async_copy(tab_ref.at[idxr_v.at[c + 1]], rows_r.at[bnxt], sem_nxt)
            pltpu.async_copy(tab_ref.at[idxs_v.at[c + 1]], rows_s.at[bnxt], sem_nxt)

        def vrow(r, cc):
            for t in range(h // 16):
                sl = pl.ds(t * 16, 16)
                rows_r[bcur, r, sl] = rows_r[bcur, r, sl] + rows_s[bcur, r, sl]
            return cc

        lax.fori_loop(0, CHUNK, vrow, 0)
        pltpu.sync_copy(rows_r.at[bcur],
                        out_ref.at[pl.ds((start + c) * CHUNK, CHUNK)])

    def pair(p, carry):
        c = 2 * p
        step(c, 0, sem_a, sem_b, True)
        step(c + 1, 1, sem_b, sem_a, True)
        return carry

    lax.fori_loop(0, cnt // 2 - 1, pair, 0)
    c = cnt - 2
    step(c, 0, sem_a, sem_b, True)
    step(c + 1, 1, sem_b, sem_a, False)


def _sc_gather(table, ridx2d, sidx2d):
    h = table.shape[1]
    nblk = ridx2d.shape[0]
    cnt = nblk // NW
    assert nblk == NW * cnt and cnt % 8 == 0
    mesh = plsc.VectorSubcoreMesh(
        core_axis_name="c", subcore_axis_name="s", num_cores=NC, num_subcores=NS
    )
    f = pl.kernel(
        functools.partial(_gather_body, cnt),
        out_type=jax.ShapeDtypeStruct((nblk * CHUNK, h), jnp.float32),
        mesh=mesh,
        scratch_types=[
            pltpu.VMEM((cnt, CHUNK), jnp.int32),
            pltpu.VMEM((cnt, CHUNK), jnp.int32),
            pltpu.VMEM((2, CHUNK, h), jnp.float32),
            pltpu.VMEM((2, CHUNK, h), jnp.float32),
            pltpu.SemaphoreType.DMA,
            pltpu.SemaphoreType.DMA,
        ],
    )
    return f(table, ridx2d, sidx2d)


def _c_body(ea_ref, g_ref, w_ref, aux_ref, upd_ref):
    ea = ea_ref[...]
    aux = aux_ref[...]
    h = jnp.dot(ea, w_ref[0], preferred_element_type=jnp.float32)
    h = jnp.maximum(h + g_ref[...] + aux[0], 0.0)
    h = jnp.maximum(jnp.dot(h, w_ref[1], preferred_element_type=jnp.float32) + aux[1], 0.0)
    h = jnp.maximum(jnp.dot(h, w_ref[2], preferred_element_type=jnp.float32) + aux[2], 0.0)
    h = jnp.dot(h, w_ref[3], preferred_element_type=jnp.float32) + aux[3]
    var = jnp.mean(h * h, axis=-1, keepdims=True)
    upd = h * lax.rsqrt(var + 1e-06) * (1.0 + aux[4])
    upd_ref[...] = upd


def _stage_c(edge_attr, gathered, w_e, aux_e, e_pad):
    e, h = edge_attr.shape
    nb = e // EDGE_BLK
    return pl.pallas_call(
        _c_body,
        grid=(nb,),
        in_specs=[
            pl.BlockSpec((EDGE_BLK, h), lambda i: (i, 0)),
            pl.BlockSpec((EDGE_BLK, h), lambda i: (i, 0)),
            pl.BlockSpec((4, h, h), lambda i: (0, 0, 0)),
            pl.BlockSpec((8, h), lambda i: (0, 0)),
        ],
        out_specs=pl.BlockSpec((EDGE_BLK, h), lambda i: (i, 0)),
        out_shape=jax.ShapeDtypeStruct((e_pad, h), jnp.float32),
    )(edge_attr, gathered, w_e, aux_e)


def _c2_body(ea_ref, upd_ref, eo_ref):
    eo_ref[...] = ea_ref[...] + upd_ref[...]


def _stage_c2(edge_attr, upd):
    e, h = edge_attr.shape
    nb = e // EDGE_BLK
    return pl.pallas_call(
        _c2_body,
        grid=(nb,),
        in_specs=[
            pl.BlockSpec((EDGE_BLK, h), lambda i: (i, 0)),
            pl.BlockSpec((EDGE_BLK, h), lambda i: (i, 0)),
        ],
        out_specs=pl.BlockSpec((EDGE_BLK, h), lambda i: (i, 0)),
        out_shape=jax.ShapeDtypeStruct((e, h), jnp.float32),
    )(edge_attr, upd)


def _scatter_body(n_pad, nblk, upd_ref, ridx_ref, out_ref,
                  idx_v, rows2, aggr, sem_a, sem_b):
    h = rows2.shape[2]
    cid = lax.axis_index("c")
    sid = lax.axis_index("s")
    wid = sid * NC + cid
    rows_per_tile = n_pad // NS

    zeros16 = jnp.zeros((16,), jnp.float32)

    def zrow(r, carry):
        for t in range(h // 16):
            rows2[0, r, pl.ds(t * 16, 16)] = zeros16
        return carry

    lax.fori_loop(0, CHUNK, zrow, 0)
    zbase = sid * rows_per_tile
    nfull = rows_per_tile // CHUNK
    rem = rows_per_tile % CHUNK
    for z in range(nfull):
        pltpu.sync_copy(rows2.at[0], aggr.at[pl.ds(zbase + z * CHUNK, CHUNK)])
    if rem:
        pltpu.sync_copy(rows2.at[0, pl.ds(0, rem)],
                        aggr.at[pl.ds(zbase + nfull * CHUNK, rem)])
    plsc.subcore_barrier()

    cnt = nblk // NW
    start = pl.multiple_of(wid * cnt, 8)

    pltpu.sync_copy(ridx_ref.at[pl.ds(start, cnt)], idx_v)

    pltpu.async_copy(upd_ref.at[pl.ds(start * CHUNK, CHUNK)], rows2.at[0], sem_a)

    def step(c, bcur, sem_cur, sem_nxt, issue_next):
        if issue_next:
            pltpu.async_copy(upd_ref.at[pl.ds((start + c + 1) * CHUNK, CHUNK)],
                             rows2.at[1 - bcur], sem_nxt)
        pltpu.make_async_copy(upd_ref.at[pl.ds(0, CHUNK)],
                              rows2.at[bcur], sem_cur).wait()
        pltpu.sync_copy(rows2.at[bcur], aggr.at[idx_v.at[c]], add=True)

    def pair(p, carry):
        c = 2 * p
        step(c, 0, sem_a, sem_b, True)
        step(c + 1, 1, sem_b, sem_a, True)
        return carry

    lax.fori_loop(0, cnt // 2 - 1, pair, 0)
    c = cnt - 2
    step(c, 0, sem_a, sem_b, True)
    step(c + 1, 1, sem_b, sem_a, False)
    plsc.subcore_barrier()
    pltpu.sync_copy(
        aggr.at[pl.ds(sid * rows_per_tile, rows_per_tile)],
        out_ref.at[cid, pl.ds(sid * rows_per_tile, rows_per_tile)],
    )


def _sc_scatter(upd, ridx2d, n_pad):
    e, h = upd.shape
    nblk = ridx2d.shape[0]
    mesh = plsc.VectorSubcoreMesh(
        core_axis_name="c", subcore_axis_name="s", num_cores=NC, num_subcores=NS
    )
    f = pl.kernel(
        functools.partial(_scatter_body, n_pad, nblk),
        out_type=jax.ShapeDtypeStruct((NC, n_pad, h), jnp.float32),
        mesh=mesh,
        scratch_types=[
            pltpu.VMEM((nblk // NW, CHUNK), jnp.int32),
            pltpu.VMEM((2, CHUNK, h), jnp.float32),
            pltpu.VMEM_SHARED((n_pad, h), jnp.float32),
            pltpu.SemaphoreType.DMA,
            pltpu.SemaphoreType.DMA,
        ],
    )
    return f(upd, ridx2d)


def _e_body(x_ref, p0_ref, p1_ref, w_ref, aux_ref, o_ref):
    xv = x_ref[...]
    aux = aux_ref[...]
    a = p0_ref[0] + p1_ref[0]
    h = jnp.dot(xv, w_ref[0], preferred_element_type=jnp.float32)
    h = h + jnp.dot(a, w_ref[1], preferred_element_type=jnp.float32)
    h = jnp.maximum(h + aux[0], 0.0)
    h = jnp.maximum(jnp.dot(h, w_ref[2], preferred_element_type=jnp.float32) + aux[1], 0.0)
    h = jnp.maximum(jnp.dot(h, w_ref[3], preferred_element_type=jnp.float32) + aux[2], 0.0)
    h = jnp.dot(h, w_ref[4], preferred_element_type=jnp.float32) + aux[3]
    var = jnp.mean(h * h, axis=-1, keepdims=True)
    upd = h * lax.rsqrt(var + 1e-06) * (1.0 + aux[4])
    o_ref[...] = xv + upd


def _stage_e(x, parts, w_n, aux_n):
    n, h = x.shape
    nb = n // NODE_BLK
    return pl.pallas_call(
        _e_body,
        grid=(nb,),
        in_specs=[
            pl.BlockSpec((NODE_BLK, h), lambda i: (i, 0)),
            pl.BlockSpec((1, NODE_BLK, h), lambda i: (0, i, 0)),
            pl.BlockSpec((1, NODE_BLK, h), lambda i: (1, i, 0)),
            pl.BlockSpec((5, h, h), lambda i: (0, 0, 0)),
            pl.BlockSpec((8, h), lambda i: (0, 0)),
        ],
        out_specs=pl.BlockSpec((NODE_BLK, h), lambda i: (i, 0)),
        out_shape=jax.ShapeDtypeStruct((n, h), jnp.float32),
    )(x, parts, parts, w_n, aux_n)


def _aux_stack(h, biases, scale):
    aux = jnp.zeros((8, h), jnp.float32)
    for i, b in enumerate(biases):
        aux = aux.at[i].set(b)
    return aux.at[4].set(scale)


def kernel(x, edge_attr, senders, receivers, edge_Ws, edge_bs, edge_scale,
           node_Ws, node_bs, node_scale):
    n, h = x.shape
    e = edge_attr.shape[0]
    recv = receivers.astype(jnp.int32)
    send = senders.astype(jnp.int32)

    w1 = edge_Ws[0]
    w_rs = jnp.stack([w1[h:2 * h], w1[2 * h:3 * h]])
    table = _stage_a(x, w_rs)

    nblk = e // CHUNK
    nblk_pad = _pad_blocks(nblk) * NW
    e_pad = nblk_pad * CHUNK
    pad_idx = jnp.arange(e_pad - e, dtype=jnp.int32) % n
    ridx2d = jnp.concatenate([recv, pad_idx]).reshape(nblk_pad, CHUNK)
    sidx2d = jnp.concatenate([send + n, n + pad_idx]).reshape(nblk_pad, CHUNK)
    gathered = _sc_gather(table, ridx2d, sidx2d)

    w_e = jnp.stack([w1[:h], edge_Ws[1], edge_Ws[2], edge_Ws[3]])
    aux_e = _aux_stack(h, edge_bs, edge_scale)
    upd = _stage_c(edge_attr, gathered, w_e, aux_e, e_pad)
    edge_out = _stage_c2(edge_attr, upd)

    n_pad = ((n + 16 * 8 - 1) // (16 * 8)) * (16 * 8)
    dummy = n + jnp.arange(e_pad - e, dtype=jnp.int32) % (n_pad - n)
    sidx_scat = jnp.concatenate([recv, dummy]).reshape(nblk_pad, CHUNK)
    parts = _sc_scatter(upd, sidx_scat, n_pad)

    w_n = jnp.stack([node_Ws[0][:h], node_Ws[0][h:], node_Ws[1], node_Ws[2], node_Ws[3]])
    aux_n = _aux_stack(h, node_bs, node_scale)
    x_out = _stage_e(x, parts, w_n, aux_n)
    return (x_out, edge_out)


# speedup vs baseline: 1.3114x; 1.3114x over previous
import functools

import jax
import jax.numpy as jnp
from jax import lax
from jax.experimental import pallas as pl
from jax.experimental.pallas import tpu as pltpu
from jax.experimental.pallas import tpu_sc as plsc

NC = 2
NS = 16
NW = NC * NS
CHUNK = 128

NODE_BLK = 2000
EDGE_BLK = 2560


def _a_body(x_ref, w_ref, o_ref):
    o_ref[...] = jnp.dot(x_ref[...], w_ref[0], preferred_element_type=jnp.float32)


def _stage_a(x, w1_3d):
    n, h = x.shape
    nb = n // NODE_BLK
    return pl.pallas_call(
        _a_body,
        grid=(2, nb),
        in_specs=[
            pl.BlockSpec((NODE_BLK, h), lambda j, i: (i, 0)),
            pl.BlockSpec((1, h, h), lambda j, i: (j + 1, 0, 0)),
        ],
        out_specs=pl.BlockSpec((NODE_BLK, h), lambda j, i: (j * nb + i, 0)),
        out_shape=jax.ShapeDtypeStruct((2 * n, h), jnp.float32),
    )(x, w1_3d)


def _pad_blocks(nblk):
    per_tile = -(-nblk // (NW * 8)) * 8
    return per_tile


def _gather_body(cnt, tab_ref, ridx_ref, out_ref,
                 idxr_v, idxs_v, rows_r, rows_s, sem_a, sem_b):
    h = tab_ref.shape[1]
    wid = lax.axis_index("s") * NC + lax.axis_index("c")
    start = pl.multiple_of(wid * cnt, 8)

    pltpu.sync_copy(ridx_ref.at[0, pl.ds(start, cnt)], idxr_v)
    pltpu.sync_copy(ridx_ref.at[1, pl.ds(start, cnt)], idxs_v)

    pltpu.async_copy(tab_ref.at[idxr_v.at[0]], rows_r.at[0], sem_a)
    pltpu.async_copy(tab_ref.at[idxs_v.at[0]], rows_s.at[0], sem_a)

    def step(c, bcur, sem_cur, sem_nxt, issue_next):
        bnxt = 1 - bcur

        pltpu.make_async_copy(tab_ref.at[idxr_v.at[0]], rows_r.at[bcur], sem_cur).wait()
        pltpu.make_async_copy(tab_ref.at[idxs_v.at[0]], rows_s.at[bcur], sem_cur).wait()

        if issue_next:
            pltpu.async_copy(tab_ref.at[idxr_v.at[c + 1]], rows_r.at[bnxt], sem_nxt)
            pltpu.async_copy(tab_ref.at[idxs_v.at[c + 1]], rows_s.at[bnxt], sem_nxt)

        def vrow(r, cc):
            for t in range(h // 16):
                sl = pl.ds(t * 16, 16)
                rows_r[bcur, r, sl] = rows_r[bcur, r, sl] + rows_s[bcur, r, sl]
            return cc

        lax.fori_loop(0, CHUNK, vrow, 0)
        pltpu.sync_copy(rows_r.at[bcur],
                        out_ref.at[pl.ds((start + c) * CHUNK, CHUNK)])

    def pair(p, carry):
        c = 2 * p
        step(c, 0, sem_a, sem_b, True)
        step(c + 1, 1, sem_b, sem_a, True)
        return carry

    lax.fori_loop(0, cnt // 2 - 1, pair, 0)
    c = cnt - 2
    step(c, 0, sem_a, sem_b, True)
    step(c + 1, 1, sem_b, sem_a, False)


def _sc_gather(table, idx3):
    h = table.shape[1]
    nblk = idx3.shape[1]
    cnt = nblk // NW
    assert nblk == NW * cnt and cnt % 8 == 0
    mesh = plsc.VectorSubcoreMesh(
        core_axis_name="c", subcore_axis_name="s", num_cores=NC, num_subcores=NS
    )
    f = pl.kernel(
        functools.partial(_gather_body, cnt),
        out_type=jax.ShapeDtypeStruct((nblk * CHUNK, h), jnp.float32),
        mesh=mesh,
        scratch_types=[
            pltpu.VMEM((cnt, CHUNK), jnp.int32),
            pltpu.VMEM((cnt, CHUNK), jnp.int32),
            pltpu.VMEM((2, CHUNK, h), jnp.float32),
            pltpu.VMEM((2, CHUNK, h), jnp.float32),
            pltpu.SemaphoreType.DMA,
            pltpu.SemaphoreType.DMA,
        ],
    )
    return f(table, idx3)


def _c_body(ea_ref, g_ref, w1_ref, w2_ref, w3_ref, w4_ref,
            b1_ref, b2_ref, b3_ref, b4_ref, sc_ref, upd_ref, eo_ref):
    ea = ea_ref[...]
    h = jnp.dot(ea, w1_ref[0], preferred_element_type=jnp.float32)
    h = jnp.maximum(h + g_ref[...] + b1_ref[...], 0.0)
    h = jnp.maximum(jnp.dot(h, w2_ref[...], preferred_element_type=jnp.float32) + b2_ref[...], 0.0)
    h = jnp.maximum(jnp.dot(h, w3_ref[...], preferred_element_type=jnp.float32) + b3_ref[...], 0.0)
    h = jnp.dot(h, w4_ref[...], preferred_element_type=jnp.float32) + b4_ref[...]
    var = jnp.mean(h * h, axis=-1, keepdims=True)
    upd = h * lax.rsqrt(var + 1e-06) * (1.0 + sc_ref[...])
    upd_ref[...] = upd
    eo_ref[...] = ea + upd


def _stage_c(edge_attr, gathered, w1_3d, ws, bs, scale, e_pad):
    e, h = edge_attr.shape
    nb = e // EDGE_BLK
    full_w = pl.BlockSpec((h, h), lambda i: (0, 0))
    row = pl.BlockSpec((1, h), lambda i: (0, 0))
    return pl.pallas_call(
        _c_body,
        grid=(nb,),
        in_specs=[
            pl.BlockSpec((EDGE_BLK, h), lambda i: (i, 0)),
            pl.BlockSpec((EDGE_BLK, h), lambda i: (i, 0)),
            pl.BlockSpec((1, h, h), lambda i: (0, 0, 0)),
            full_w, full_w, full_w,
            row, row, row, row, row,
        ],
        out_specs=[
            pl.BlockSpec((EDGE_BLK, h), lambda i: (i, 0)),
            pl.BlockSpec((EDGE_BLK, h), lambda i: (i, 0)),
        ],
        out_shape=[
            jax.ShapeDtypeStruct((e_pad, h), jnp.float32),
            jax.ShapeDtypeStruct((e, h), jnp.float32),
        ],
    )(edge_attr, gathered, w1_3d, ws[0], ws[1], ws[2],
      bs[0].reshape(1, h), bs[1].reshape(1, h), bs[2].reshape(1, h),
      bs[3].reshape(1, h), scale.reshape(1, h))


def _scatter_body(n_pad, nblk, upd_ref, ridx_ref, out_ref,
                  idx_v, rows2, aggr, sem_a, sem_b):
    h = rows2.shape[2]
    cid = lax.axis_index("c")
    sid = lax.axis_index("s")
    wid = sid * NC + cid
    rows_per_tile = n_pad // NS

    zeros16 = jnp.zeros((16,), jnp.float32)

    def zrow(r, carry):
        for t in range(h // 16):
            rows2[0, r, pl.ds(t * 16, 16)] = zeros16
        return carry

    lax.fori_loop(0, CHUNK, zrow, 0)
    zbase = sid * rows_per_tile
    nfull = rows_per_tile // CHUNK
    rem = rows_per_tile % CHUNK
    for z in range(nfull):
        pltpu.sync_copy(rows2.at[0], aggr.at[pl.ds(zbase + z * CHUNK, CHUNK)])
    if rem:
        pltpu.sync_copy(rows2.at[0, pl.ds(0, rem)],
                        aggr.at[pl.ds(zbase + nfull * CHUNK, rem)])
    plsc.subcore_barrier()

    cnt = nblk // NW
    start = pl.multiple_of(wid * cnt, 8)

    pltpu.sync_copy(ridx_ref.at[2, pl.ds(start, cnt)], idx_v)

    pltpu.async_copy(upd_ref.at[pl.ds(start * CHUNK, CHUNK)], rows2.at[0], sem_a)

    def step(c, bcur, sem_cur, sem_nxt, issue_next):
        if issue_next:
            pltpu.async_copy(upd_ref.at[pl.ds((start + c + 1) * CHUNK, CHUNK)],
                             rows2.at[1 - bcur], sem_nxt)
        pltpu.make_async_copy(upd_ref.at[pl.ds(0, CHUNK)],
                              rows2.at[bcur], sem_cur).wait()
        pltpu.sync_copy(rows2.at[bcur], aggr.at[idx_v.at[c]], add=True)

    def pair(p, carry):
        c = 2 * p
        step(c, 0, sem_a, sem_b, True)
        step(c + 1, 1, sem_b, sem_a, True)
        return carry

    lax.fori_loop(0, cnt // 2 - 1, pair, 0)
    c = cnt - 2
    step(c, 0, sem_a, sem_b, True)
    step(c + 1, 1, sem_b, sem_a, False)
    plsc.subcore_barrier()
    pltpu.sync_copy(
        aggr.at[pl.ds(sid * rows_per_tile, rows_per_tile)],
        out_ref.at[cid, pl.ds(sid * rows_per_tile, rows_per_tile)],
    )


def _sc_scatter(upd, idx3, n_pad):
    e, h = upd.shape
    nblk = idx3.shape[1]
    mesh = plsc.VectorSubcoreMesh(
        core_axis_name="c", subcore_axis_name="s", num_cores=NC, num_subcores=NS
    )
    f = pl.kernel(
        functools.partial(_scatter_body, n_pad, nblk),
        out_type=jax.ShapeDtypeStruct((NC, n_pad, h), jnp.float32),
        mesh=mesh,
        scratch_types=[
            pltpu.VMEM((nblk // NW, CHUNK), jnp.int32),
            pltpu.VMEM((2, CHUNK, h), jnp.float32),
            pltpu.VMEM_SHARED((n_pad, h), jnp.float32),
            pltpu.SemaphoreType.DMA,
            pltpu.SemaphoreType.DMA,
        ],
    )
    return f(upd, idx3)


def _e_body(x_ref, p0_ref, p1_ref, w1_ref, w2_ref, w3_ref, w4_ref,
            b1_ref, b2_ref, b3_ref, b4_ref, sc_ref, o_ref):
    xv = x_ref[...]
    a = p0_ref[0] + p1_ref[0]
    h = jnp.dot(xv, w1_ref[0], preferred_element_type=jnp.float32)
    h = h + jnp.dot(a, w1_ref[1], preferred_element_type=jnp.float32)
    h = jnp.maximum(h + b1_ref[...], 0.0)
    h = jnp.maximum(jnp.dot(h, w2_ref[...], preferred_element_type=jnp.float32) + b2_ref[...], 0.0)
    h = jnp.maximum(jnp.dot(h, w3_ref[...], preferred_element_type=jnp.float32) + b3_ref[...], 0.0)
    h = jnp.dot(h, w4_ref[...], preferred_element_type=jnp.float32) + b4_ref[...]
    var = jnp.mean(h * h, axis=-1, keepdims=True)
    upd = h * lax.rsqrt(var + 1e-06) * (1.0 + sc_ref[...])
    o_ref[...] = xv + upd


def _stage_e(x, parts, wn1_3d, ws, bs, scale):
    n, h = x.shape
    nb = n // NODE_BLK
    full_w = pl.BlockSpec((h, h), lambda i: (0, 0))
    row = pl.BlockSpec((1, h), lambda i: (0, 0))
    return pl.pallas_call(
        _e_body,
        grid=(nb,),
        in_specs=[
            pl.BlockSpec((NODE_BLK, h), lambda i: (i, 0)),
            pl.BlockSpec((1, NODE_BLK, h), lambda i: (0, i, 0)),
            pl.BlockSpec((1, NODE_BLK, h), lambda i: (1, i, 0)),
            pl.BlockSpec((2, h, h), lambda i: (0, 0, 0)),
            full_w, full_w, full_w,
            row, row, row, row, row,
        ],
        out_specs=pl.BlockSpec((NODE_BLK, h), lambda i: (i, 0)),
        out_shape=jax.ShapeDtypeStruct((n, h), jnp.float32),
    )(x, parts, parts, wn1_3d, ws[0], ws[1], ws[2],
      bs[0].reshape(1, h), bs[1].reshape(1, h), bs[2].reshape(1, h),
      bs[3].reshape(1, h), scale.reshape(1, h))


def kernel(x, edge_attr, senders, receivers, edge_Ws, edge_bs, edge_scale,
           node_Ws, node_bs, node_scale):
    n, h = x.shape
    e = edge_attr.shape[0]
    recv = receivers.astype(jnp.int32)
    send = senders.astype(jnp.int32)

    w1_3d = edge_Ws[0].reshape(3, h, h)
    table = _stage_a(x, w1_3d)

    nblk = e // CHUNK
    nblk_pad = _pad_blocks(nblk) * NW
    e_pad = nblk_pad * CHUNK
    n_pad = ((n + 16 * 8 - 1) // (16 * 8)) * (16 * 8)
    pad_idx = jnp.arange(e_pad - e, dtype=jnp.int32) % n
    dummy = n + jnp.arange(e_pad - e, dtype=jnp.int32) % (n_pad - n)
    idx3 = jnp.concatenate(
        [recv, pad_idx, send + n, n + pad_idx, recv, dummy]
    ).reshape(3, nblk_pad, CHUNK)
    gathered = _sc_gather(table, idx3)

    upd, edge_out = _stage_c(edge_attr, gathered, w1_3d, edge_Ws[1:],
                             edge_bs, edge_scale, e_pad)
    parts = _sc_scatter(upd, idx3, n_pad)

    x_out = _stage_e(x, parts, node_Ws[0].reshape(2, h, h), node_Ws[1:],
                     node_bs, node_scale)
    return (x_out, edge_out)


# speedup vs baseline: 1.3322x; 1.0158x over previous
import functools

import jax
import jax.numpy as jnp
from jax import lax
from jax.experimental import pallas as pl
from jax.experimental.pallas import tpu as pltpu
from jax.experimental.pallas import tpu_sc as plsc

NC = 2
NS = 16
NW = NC * NS
CHUNK = 128

NODE_BLK = 2000
EDGE_BLK = 2560


def _a_body(x_ref, w_ref, o_ref):
    o_ref[...] = jnp.dot(x_ref[...], w_ref[0], preferred_element_type=jnp.float32)


def _stage_a(x, w1_3d):
    n, h = x.shape
    nb = n // NODE_BLK
    return pl.pallas_call(
        _a_body,
        grid=(2, nb),
        in_specs=[
            pl.BlockSpec((NODE_BLK, h), lambda j, i: (i, 0)),
            pl.BlockSpec((1, h, h), lambda j, i: (j + 1, 0, 0)),
        ],
        out_specs=pl.BlockSpec((NODE_BLK, h), lambda j, i: (j * nb + i, 0)),
        out_shape=jax.ShapeDtypeStruct((2 * n, h), jnp.float32),
    )(x, w1_3d)


def _pad_blocks(nblk):
    per_tile = -(-nblk // (NW * 8)) * 8
    return per_tile


def _gather_body(cnt, tab_ref, ridx_ref, out_ref,
                 idxr_v, idxs_v, rows_r, rows_s, sem_a, sem_b):
    h = tab_ref.shape[1]
    wid = lax.axis_index("s") * NC + lax.axis_index("c")
    start = pl.multiple_of(wid * cnt, 8)

    pltpu.sync_copy(ridx_ref.at[0, pl.ds(start, cnt)], idxr_v)
    pltpu.sync_copy(ridx_ref.at[1, pl.ds(start, cnt)], idxs_v)

    pltpu.async_copy(tab_ref.at[idxr_v.at[0]], rows_r.at[0], sem_a)
    pltpu.async_copy(tab_ref.at[idxs_v.at[0]], rows_s.at[0], sem_a)

    def step(c, bcur, sem_cur, sem_nxt, issue_next):
        bnxt = 1 - bcur

        if issue_next:
            pltpu.async_copy(tab_ref.at[idxr_v.at[c + 1]], rows_r.at[bnxt], sem_nxt)
            pltpu.async_copy(tab_ref.at[idxs_v.at[c + 1]], rows_s.at[bnxt], sem_nxt)

        pltpu.make_async_copy(tab_ref.at[idxr_v.at[0]], rows_r.at[bcur], sem_cur).wait()
        pltpu.make_async_copy(tab_ref.at[idxs_v.at[0]], rows_s.at[bcur], sem_cur).wait()

        def vrow(r, cc):
            for t in range(h // 16):
                sl = pl.ds(t * 16, 16)
                rows_r[bcur, r, sl] = rows_r[bcur, r, sl] + rows_s[bcur, r, sl]
            return cc

        lax.fori_loop(0, CHUNK, vrow, 0)
        pltpu.sync_copy(rows_r.at[bcur],
                        out_ref.at[pl.ds((start + c) * CHUNK, CHUNK)])

    def pair(p, carry):
        c = 2 * p
        step(c, 0, sem_a, sem_b, True)
        step(c + 1, 1, sem_b, sem_a, True)
        return carry

    lax.fori_loop(0, cnt // 2 - 1, pair, 0)
    c = cnt - 2
    step(c, 0, sem_a, sem_b, True)
    step(c + 1, 1, sem_b, sem_a, False)


def _sc_gather(table, idx3):
    h = table.shape[1]
    nblk = idx3.shape[1]
    cnt = nblk // NW
    assert nblk == NW * cnt and cnt % 8 == 0
    mesh = plsc.VectorSubcoreMesh(
        core_axis_name="c", subcore_axis_name="s", num_cores=NC, num_subcores=NS
    )
    f = pl.kernel(
        functools.partial(_gather_body, cnt),
        out_type=jax.ShapeDtypeStruct((nblk * CHUNK, h), jnp.float32),
        mesh=mesh,
        scratch_types=[
            pltpu.VMEM((cnt, CHUNK), jnp.int32),
            pltpu.VMEM((cnt, CHUNK), jnp.int32),
            pltpu.VMEM((2, CHUNK, h), jnp.float32),
            pltpu.VMEM((2, CHUNK, h), jnp.float32),
            pltpu.SemaphoreType.DMA,
            pltpu.SemaphoreType.DMA,
        ],
    )
    return f(table, idx3)


def _c_body(ea_ref, g_ref, w1_ref, w2_ref, w3_ref, w4_ref,
            b1_ref, b2_ref, b3_ref, b4_ref, sc_ref, upd_ref, eo_ref):
    ea = ea_ref[...]
    h = jnp.dot(ea, w1_ref[0], preferred_element_type=jnp.float32)
    h = jnp.maximum(h + g_ref[...] + b1_ref[...], 0.0)
    h = jnp.maximum(jnp.dot(h, w2_ref[...], preferred_element_type=jnp.float32) + b2_ref[...], 0.0)
    h = jnp.maximum(jnp.dot(h, w3_ref[...], preferred_element_type=jnp.float32) + b3_ref[...], 0.0)
    h = jnp.dot(h, w4_ref[...], preferred_element_type=jnp.float32) + b4_ref[...]
    var = jnp.mean(h * h, axis=-1, keepdims=True)
    upd = h * lax.rsqrt(var + 1e-06) * (1.0 + sc_ref[...])
    upd_ref[...] = upd
    eo_ref[...] = ea + upd


def _stage_c(edge_attr, gathered, w1_3d, ws, bs, scale, e_pad):
    e, h = edge_attr.shape
    nb = e // EDGE_BLK
    full_w = pl.BlockSpec((h, h), lambda i: (0, 0))
    row = pl.BlockSpec((1, h), lambda i: (0, 0))
    return pl.pallas_call(
        _c_body,
        grid=(nb,),
        in_specs=[
            pl.BlockSpec((EDGE_BLK, h), lambda i: (i, 0)),
            pl.BlockSpec((EDGE_BLK, h), lambda i: (i, 0)),
            pl.BlockSpec((1, h, h), lambda i: (0, 0, 0)),
            full_w, full_w, full_w,
            row, row, row, row, row,
        ],
        out_specs=[
            pl.BlockSpec((EDGE_BLK, h), lambda i: (i, 0)),
            pl.BlockSpec((EDGE_BLK, h), lambda i: (i, 0)),
        ],
        out_shape=[
            jax.ShapeDtypeStruct((e_pad, h), jnp.float32),
            jax.ShapeDtypeStruct((e, h), jnp.float32),
        ],
    )(edge_attr, gathered, w1_3d, ws[0], ws[1], ws[2],
      bs[0].reshape(1, h), bs[1].reshape(1, h), bs[2].reshape(1, h),
      bs[3].reshape(1, h), scale.reshape(1, h))


def _scatter_body(n_pad, nblk, upd_ref, ridx_ref, out_ref,
                  idx_v, rows2, aggr, sem_a, sem_b):
    h = rows2.shape[2]
    cid = lax.axis_index("c")
    sid = lax.axis_index("s")
    wid = sid * NC + cid
    rows_per_tile = n_pad // NS

    zeros16 = jnp.zeros((16,), jnp.float32)

    def zrow(r, carry):
        for t in range(h // 16):
            rows2[0, r, pl.ds(t * 16, 16)] = zeros16
        return carry

    lax.fori_loop(0, CHUNK, zrow, 0)
    zbase = sid * rows_per_tile
    nfull = rows_per_tile // CHUNK
    rem = rows_per_tile % CHUNK
    for z in range(nfull):
        pltpu.sync_copy(rows2.at[0], aggr.at[pl.ds(zbase + z * CHUNK, CHUNK)])
    if rem:
        pltpu.sync_copy(rows2.at[0, pl.ds(0, rem)],
                        aggr.at[pl.ds(zbase + nfull * CHUNK, rem)])
    plsc.subcore_barrier()

    cnt = nblk // NW
    start = pl.multiple_of(wid * cnt, 8)

    pltpu.sync_copy(ridx_ref.at[2, pl.ds(start, cnt)], idx_v)

    pltpu.async_copy(upd_ref.at[pl.ds(start * CHUNK, CHUNK)], rows2.at[0], sem_a)

    def step(c, bcur, sem_cur, sem_nxt, issue_next):
        if issue_next:
            pltpu.async_copy(upd_ref.at[pl.ds((start + c + 1) * CHUNK, CHUNK)],
                             rows2.at[1 - bcur], sem_nxt)
        pltpu.make_async_copy(upd_ref.at[pl.ds(0, CHUNK)],
                              rows2.at[bcur], sem_cur).wait()
        pltpu.sync_copy(rows2.at[bcur], aggr.at[idx_v.at[c]], add=True)

    def pair(p, carry):
        c = 2 * p
        step(c, 0, sem_a, sem_b, True)
        step(c + 1, 1, sem_b, sem_a, True)
        return carry

    lax.fori_loop(0, cnt // 2 - 1, pair, 0)
    c = cnt - 2
    step(c, 0, sem_a, sem_b, True)
    step(c + 1, 1, sem_b, sem_a, False)
    plsc.subcore_barrier()
    pltpu.sync_copy(
        aggr.at[pl.ds(sid * rows_per_tile, rows_per_tile)],
        out_ref.at[cid, pl.ds(sid * rows_per_tile, rows_per_tile)],
    )


def _sc_scatter(upd, idx3, n_pad):
    e, h = upd.shape
    nblk = idx3.shape[1]
    mesh = plsc.VectorSubcoreMesh(
        core_axis_name="c", subcore_axis_name="s", num_cores=NC, num_subcores=NS
    )
    f = pl.kernel(
        functools.partial(_scatter_body, n_pad, nblk),
        out_type=jax.ShapeDtypeStruct((NC, n_pad, h), jnp.float32),
        mesh=mesh,
        scratch_types=[
            pltpu.VMEM((nblk // NW, CHUNK), jnp.int32),
            pltpu.VMEM((2, CHUNK, h), jnp.float32),
            pltpu.VMEM_SHARED((n_pad, h), jnp.float32),
            pltpu.SemaphoreType.DMA,
            pltpu.SemaphoreType.DMA,
        ],
    )
    return f(upd, idx3)


def _e_body(x_ref, p0_ref, p1_ref, w1_ref, w2_ref, w3_ref, w4_ref,
            b1_ref, b2_ref, b3_ref, b4_ref, sc_ref, o_ref):
    xv = x_ref[...]
    a = p0_ref[0] + p1_ref[0]
    h = jnp.dot(xv, w1_ref[0], preferred_element_type=jnp.float32)
    h = h + jnp.dot(a, w1_ref[1], preferred_element_type=jnp.float32)
    h = jnp.maximum(h + b1_ref[...], 0.0)
    h = jnp.maximum(jnp.dot(h, w2_ref[...], preferred_element_type=jnp.float32) + b2_ref[...], 0.0)
    h = jnp.maximum(jnp.dot(h, w3_ref[...], preferred_element_type=jnp.float32) + b3_ref[...], 0.0)
    h = jnp.dot(h, w4_ref[...], preferred_element_type=jnp.float32) + b4_ref[...]
    var = jnp.mean(h * h, axis=-1, keepdims=True)
    upd = h * lax.rsqrt(var + 1e-06) * (1.0 + sc_ref[...])
    o_ref[...] = xv + upd


def _stage_e(x, parts, wn1_3d, ws, bs, scale):
    n, h = x.shape
    nb = n // NODE_BLK
    full_w = pl.BlockSpec((h, h), lambda i: (0, 0))
    row = pl.BlockSpec((1, h), lambda i: (0, 0))
    return pl.pallas_call(
        _e_body,
        grid=(nb,),
        in_specs=[
            pl.BlockSpec((NODE_BLK, h), lambda i: (i, 0)),
            pl.BlockSpec((1, NODE_BLK, h), lambda i: (0, i, 0)),
            pl.BlockSpec((1, NODE_BLK, h), lambda i: (1, i, 0)),
            pl.BlockSpec((2, h, h), lambda i: (0, 0, 0)),
            full_w, full_w, full_w,
            row, row, row, row, row,
        ],
        out_specs=pl.BlockSpec((NODE_BLK, h), lambda i: (i, 0)),
        out_shape=jax.ShapeDtypeStruct((n, h), jnp.float32),
    )(x, parts, parts, wn1_3d, ws[0], ws[1], ws[2],
      bs[0].reshape(1, h), bs[1].reshape(1, h), bs[2].reshape(1, h),
      bs[3].reshape(1, h), scale.reshape(1, h))


def kernel(x, edge_attr, senders, receivers, edge_Ws, edge_bs, edge_scale,
           node_Ws, node_bs, node_scale):
    n, h = x.shape
    e = edge_attr.shape[0]
    recv = receivers.astype(jnp.int32)
    send = senders.astype(jnp.int32)

    w1_3d = edge_Ws[0].reshape(3, h, h)
    table = _stage_a(x, w1_3d)

    nblk = e // CHUNK
    nblk_pad = _pad_blocks(nblk) * NW
    e_pad = nblk_pad * CHUNK
    n_pad = ((n + 16 * 8 - 1) // (16 * 8)) * (16 * 8)
    pad_idx = jnp.arange(e_pad - e, dtype=jnp.int32) % n
    dummy = n + jnp.arange(e_pad - e, dtype=jnp.int32) % (n_pad - n)
    idx3 = jnp.concatenate(
        [recv, pad_idx, send + n, n + pad_idx, recv, dummy]
    ).reshape(3, nblk_pad, CHUNK)
    gathered = _sc_gather(table, idx3)

    upd, edge_out = _stage_c(edge_attr, gathered, w1_3d, edge_Ws[1:],
                             edge_bs, edge_scale, e_pad)
    parts = _sc_scatter(upd, idx3, n_pad)

    x_out = _stage_e(x, parts, node_Ws[0].reshape(2, h, h), node_Ws[1:],
                     node_bs, node_scale)
    return (x_out, edge_out)
